# Initial kernel scaffold; baseline (speedup 1.0000x reference)
import jax
import jax.numpy as jnp
from jax.experimental import pallas as pl


def kernel(queries, keys, values):
    raise NotImplementedError("write your pallas kernel here")


# speedup vs baseline: 2.3394x; 2.3394x over previous
import math

import numpy as np
import jax
import jax.numpy as jnp
from jax.experimental import pallas as pl
from jax.experimental.pallas import tpu as pltpu

L = 2048
H = 16
E = 64
SAMPLE_K = 40
N_TOP = 40
SCALE = 1.0 / math.sqrt(E)
KT = 512
BT = 256

_HI = jax.lax.Precision.HIGHEST


def _sample_counts() -> np.ndarray:
    cpu = jax.local_devices(backend="cpu")[0]
    with jax.default_device(cpu):
        idx = np.asarray(
            jax.random.randint(jax.random.key(42), (L, SAMPLE_K), 0, L))
    cnt = np.zeros((L, L), dtype=np.int8)
    np.add.at(cnt, (np.arange(L)[:, None], idx), 1)
    return cnt


_COUNTS = _sample_counts()


def _body(q_ref, k_ref, v_ref, c_ref, o_ref):
    q = q_ref[:, 0, 0, :]
    k = k_ref[:, 0, 0, :]
    v = v_ref[:, 0, 0, :]

    qb = q.astype(jnp.bfloat16)
    run_max = jnp.full((L, 1), -jnp.inf, dtype=jnp.float32)
    run_sum = jnp.zeros((L, 1), dtype=jnp.float32)
    for t in range(L // KT):
        kt = k[t * KT:(t + 1) * KT, :].astype(jnp.bfloat16)
        st = jax.lax.dot_general(qb, kt, (((1,), (1,)), ((), ())),
                                 preferred_element_type=jnp.float32)
        ct = c_ref[:, t * KT:(t + 1) * KT]
        cf = ct.astype(jnp.float32)
        run_sum = run_sum + jnp.sum(st * cf, axis=1, keepdims=True)
        masked = jnp.where(cf > 0.0, st, -jnp.inf)
        run_max = jnp.maximum(run_max, jnp.max(masked, axis=1, keepdims=True))
    sparsity = run_max - run_sum * (1.0 / SAMPLE_K)

    iota_l = jax.lax.broadcasted_iota(jnp.int32, (L, 1), 0)
    row40 = jax.lax.broadcasted_iota(jnp.int32, (1, N_TOP), 1)
    col40 = jax.lax.broadcasted_iota(jnp.int32, (N_TOP, 1), 0)

    def step(n, carry):
        sp, ti_row, ti_col = carry
        m = jnp.max(sp)
        idx = jnp.min(jnp.where(sp == m, iota_l, L))
        sp = jnp.where(iota_l == idx, -jnp.inf, sp)
        ti_row = jnp.where(row40 == n, idx, ti_row)
        ti_col = jnp.where(col40 == n, idx, ti_col)
        return sp, ti_row, ti_col

    _, ti_row, ti_col = jax.lax.fori_loop(
        0, N_TOP, step,
        (sparsity, jnp.zeros((1, N_TOP), jnp.int32),
         jnp.zeros((N_TOP, 1), jnp.int32)))

    p = (iota_l == ti_row).astype(jnp.float32)

    q_top = jax.lax.dot_general(p, q, (((0,), (0,)), ((), ())),
                                preferred_element_type=jnp.float32,
                                precision=_HI)
    scores = jax.lax.dot_general(q_top, k, (((1,), (1,)), ((), ())),
                                 preferred_element_type=jnp.float32,
                                 precision=_HI) * SCALE
    key_pos = jax.lax.broadcasted_iota(jnp.int32, (N_TOP, L), 1)
    scores = jnp.where(key_pos > ti_col, -jnp.inf, scores)
    smax = jnp.max(scores, axis=1, keepdims=True)
    ex = jnp.exp(scores - smax)
    attn = ex / jnp.sum(ex, axis=1, keepdims=True)
    updates = jax.lax.dot_general(attn, v, (((1,), (0,)), ((), ())),
                                  preferred_element_type=jnp.float32,
                                  precision=_HI)

    ri = jax.lax.broadcasted_iota(jnp.int32, (BT, BT), 0)
    ci = jax.lax.broadcasted_iota(jnp.int32, (BT, BT), 1)
    tri = (ri >= ci).astype(jnp.float32)
    prefix = jnp.zeros((1, E), jnp.float32)
    blocks = []
    for b in range(L // BT):
        vb = v[b * BT:(b + 1) * BT, :]
        cb = jax.lax.dot_general(tri, vb, (((1,), (0,)), ((), ())),
                                 preferred_element_type=jnp.float32,
                                 precision=_HI) + prefix
        blocks.append(cb)
        prefix = cb[BT - 1:BT, :]
    ctx = jnp.concatenate(blocks, axis=0)

    scattered = jax.lax.dot_general(p, updates, (((1,), (0,)), ((), ())),
                                    preferred_element_type=jnp.float32,
                                    precision=_HI)
    is_top = jnp.sum(p, axis=1, keepdims=True) > 0.0
    o_ref[:, 0, 0, :] = jnp.where(is_top, scattered, ctx)


def kernel(queries, keys, values):
    B, Lq, Hn, En = queries.shape
    q4 = queries.reshape(L, H, 1, E)
    k4 = keys.reshape(L, H, 1, E)
    v4 = values.reshape(L, H, 1, E)
    counts = jnp.asarray(_COUNTS)

    spec = pl.BlockSpec((L, 1, 1, E), lambda h: (0, h, 0, 0))
    spec_c = pl.BlockSpec((L, L), lambda h: (0, 0))
    out = pl.pallas_call(
        _body,
        grid=(H,),
        in_specs=[spec, spec, spec, spec_c],
        out_specs=spec,
        out_shape=jax.ShapeDtypeStruct((L, H, 1, E), jnp.float32),
        compiler_params=pltpu.CompilerParams(
            dimension_semantics=("arbitrary",)),
    )(q4, k4, v4, counts)
    return out.reshape(B, Lq, Hn, En)


# speedup vs baseline: 2.3448x; 1.0023x over previous
import math

import numpy as np
import jax
import jax.numpy as jnp
from jax.experimental import pallas as pl
from jax.experimental.pallas import tpu as pltpu

L = 2048
H = 16
E = 64
SAMPLE_K = 40
N_TOP = 40
SCALE = 1.0 / math.sqrt(E)
KT = 512
BT = 256

_HI = jax.lax.Precision.HIGHEST


def _threefry2x32(k0, k1, x0, x1):
    def rotl(x, r):
        return ((x << np.uint32(r)) | (x >> np.uint32(32 - r))).astype(np.uint32)

    R = [13, 15, 26, 6, 17, 29, 16, 24]
    ks0, ks1 = np.uint32(k0), np.uint32(k1)
    ks2 = np.uint32(ks0 ^ ks1 ^ np.uint32(0x1BD11BDA))
    x0 = (x0 + ks0).astype(np.uint32)
    x1 = (x1 + ks1).astype(np.uint32)
    inject = [(ks1, ks2), (ks2, ks0), (ks0, ks1), (ks1, ks2), (ks2, ks0)]
    for g in range(5):
        for r in (R[0:4] if g % 2 == 0 else R[4:8]):
            x0 = (x0 + x1).astype(np.uint32)
            x1 = (rotl(x1, r) ^ x0).astype(np.uint32)
        a, b = inject[g]
        x0 = (x0 + a).astype(np.uint32)
        x1 = (x1 + b + np.uint32(g + 1)).astype(np.uint32)
    return x0, x1


def _sample_counts() -> np.ndarray:
    a, b = _threefry2x32(0, 42, np.zeros(2, np.uint32),
                         np.arange(2, dtype=np.uint32))
    k2 = (a[1], b[1])
    i = np.arange(L * SAMPLE_K, dtype=np.uint64)
    hi = (i >> np.uint64(32)).astype(np.uint32)
    lo = (i & np.uint64(0xFFFFFFFF)).astype(np.uint32)
    y0, y1 = _threefry2x32(k2[0], k2[1], hi, lo)
    idx = ((y0 ^ y1) % np.uint32(L)).astype(np.int32).reshape(L, SAMPLE_K)
    cnt = np.zeros((L, L), dtype=np.int8)
    np.add.at(cnt, (np.arange(L)[:, None], idx), 1)
    return cnt


_COUNTS = _sample_counts()


def _body(q_ref, k_ref, v_ref, c_ref, o_ref):
    q = q_ref[:, 0, 0, :]
    k = k_ref[:, 0, 0, :]
    v = v_ref[:, 0, 0, :]

    qb = q.astype(jnp.bfloat16)
    run_max = jnp.full((L, 1), -jnp.inf, dtype=jnp.float32)
    run_sum = jnp.zeros((L, 1), dtype=jnp.float32)
    for t in range(L // KT):
        kt = k[t * KT:(t + 1) * KT, :].astype(jnp.bfloat16)
        st = jax.lax.dot_general(qb, kt, (((1,), (1,)), ((), ())),
                                 preferred_element_type=jnp.float32)
        ct = c_ref[:, t * KT:(t + 1) * KT]
        cf = ct.astype(jnp.float32)
        run_sum = run_sum + jnp.sum(st * cf, axis=1, keepdims=True)
        masked = jnp.where(cf > 0.0, st, -jnp.inf)
        run_max = jnp.maximum(run_max, jnp.max(masked, axis=1, keepdims=True))
    sparsity = run_max - run_sum * (1.0 / SAMPLE_K)

    iota_l = jax.lax.broadcasted_iota(jnp.int32, (L, 1), 0)
    row40 = jax.lax.broadcasted_iota(jnp.int32, (1, N_TOP), 1)
    col40 = jax.lax.broadcasted_iota(jnp.int32, (N_TOP, 1), 0)

    def step(n, carry):
        sp, ti_row, ti_col = carry
        m = jnp.max(sp)
        idx = jnp.min(jnp.where(sp == m, iota_l, L))
        sp = jnp.where(iota_l == idx, -jnp.inf, sp)
        ti_row = jnp.where(row40 == n, idx, ti_row)
        ti_col = jnp.where(col40 == n, idx, ti_col)
        return sp, ti_row, ti_col

    _, ti_row, ti_col = jax.lax.fori_loop(
        0, N_TOP, step,
        (sparsity, jnp.zeros((1, N_TOP), jnp.int32),
         jnp.zeros((N_TOP, 1), jnp.int32)))

    p = (iota_l == ti_row).astype(jnp.float32)

    q_top = jax.lax.dot_general(p, q, (((0,), (0,)), ((), ())),
                                preferred_element_type=jnp.float32,
                                precision=_HI)
    scores = jax.lax.dot_general(q_top, k, (((1,), (1,)), ((), ())),
                                 preferred_element_type=jnp.float32,
                                 precision=_HI) * SCALE
    key_pos = jax.lax.broadcasted_iota(jnp.int32, (N_TOP, L), 1)
    scores = jnp.where(key_pos > ti_col, -jnp.inf, scores)
    smax = jnp.max(scores, axis=1, keepdims=True)
    ex = jnp.exp(scores - smax)
    attn = ex / jnp.sum(ex, axis=1, keepdims=True)
    updates = jax.lax.dot_general(attn, v, (((1,), (0,)), ((), ())),
                                  preferred_element_type=jnp.float32,
                                  precision=_HI)

    ri = jax.lax.broadcasted_iota(jnp.int32, (BT, BT), 0)
    ci = jax.lax.broadcasted_iota(jnp.int32, (BT, BT), 1)
    tri = (ri >= ci).astype(jnp.float32)
    prefix = jnp.zeros((1, E), jnp.float32)
    blocks = []
    for b in range(L // BT):
        vb = v[b * BT:(b + 1) * BT, :]
        cb = jax.lax.dot_general(tri, vb, (((1,), (0,)), ((), ())),
                                 preferred_element_type=jnp.float32,
                                 precision=_HI) + prefix
        blocks.append(cb)
        prefix = cb[BT - 1:BT, :]
    ctx = jnp.concatenate(blocks, axis=0)

    scattered = jax.lax.dot_general(p, updates, (((1,), (0,)), ((), ())),
                                    preferred_element_type=jnp.float32,
                                    precision=_HI)
    is_top = jnp.sum(p, axis=1, keepdims=True) > 0.0
    o_ref[:, 0, 0, :] = jnp.where(is_top, scattered, ctx)


def kernel(queries, keys, values):
    B, Lq, Hn, En = queries.shape
    q4 = queries.reshape(L, H, 1, E)
    k4 = keys.reshape(L, H, 1, E)
    v4 = values.reshape(L, H, 1, E)
    counts = jnp.asarray(_COUNTS)

    spec = pl.BlockSpec((L, 1, 1, E), lambda h: (0, h, 0, 0))
    spec_c = pl.BlockSpec((L, L), lambda h: (0, 0))
    out = pl.pallas_call(
        _body,
        grid=(H,),
        in_specs=[spec, spec, spec, spec_c],
        out_specs=spec,
        out_shape=jax.ShapeDtypeStruct((L, H, 1, E), jnp.float32),
        compiler_params=pltpu.CompilerParams(
            dimension_semantics=("arbitrary",)),
    )(q4, k4, v4, counts)
    return out.reshape(B, Lq, Hn, En)


# speedup vs baseline: 5.0549x; 2.1558x over previous
import math

import numpy as np
import jax
import jax.numpy as jnp
from jax.experimental import pallas as pl
from jax.experimental.pallas import tpu as pltpu

L = 2048
H = 16
E = 64
SAMPLE_K = 40
N_TOP = 40
SCALE = 1.0 / math.sqrt(E)
KT = 512
BT = 256


def _threefry2x32(k0, k1, x0, x1):
    def rotl(x, r):
        return ((x << np.uint32(r)) | (x >> np.uint32(32 - r))).astype(np.uint32)

    R = [13, 15, 26, 6, 17, 29, 16, 24]
    ks0, ks1 = np.uint32(k0), np.uint32(k1)
    ks2 = np.uint32(ks0 ^ ks1 ^ np.uint32(0x1BD11BDA))
    x0 = (x0 + ks0).astype(np.uint32)
    x1 = (x1 + ks1).astype(np.uint32)
    inject = [(ks1, ks2), (ks2, ks0), (ks0, ks1), (ks1, ks2), (ks2, ks0)]
    for g in range(5):
        for r in (R[0:4] if g % 2 == 0 else R[4:8]):
            x0 = (x0 + x1).astype(np.uint32)
            x1 = (rotl(x1, r) ^ x0).astype(np.uint32)
        a, b = inject[g]
        x0 = (x0 + a).astype(np.uint32)
        x1 = (x1 + b + np.uint32(g + 1)).astype(np.uint32)
    return x0, x1


def _sample_counts_t() -> np.ndarray:
    a, b = _threefry2x32(0, 42, np.zeros(2, np.uint32),
                         np.arange(2, dtype=np.uint32))
    k2 = (a[1], b[1])
    i = np.arange(L * SAMPLE_K, dtype=np.uint64)
    hi = (i >> np.uint64(32)).astype(np.uint32)
    lo = (i & np.uint64(0xFFFFFFFF)).astype(np.uint32)
    y0, y1 = _threefry2x32(k2[0], k2[1], hi, lo)
    idx = ((y0 ^ y1) % np.uint32(L)).astype(np.int32).reshape(L, SAMPLE_K)
    cnt = np.zeros((L, L), dtype=np.int8)
    np.add.at(cnt, (idx, np.arange(L)[:, None]), 1)
    return cnt


_COUNTS_T = _sample_counts_t()


def _split(x):
    hi = x.astype(jnp.bfloat16)
    lo = (x - hi.astype(jnp.float32)).astype(jnp.bfloat16)
    return hi, lo


def _mm(a, b, dims):
    return jax.lax.dot_general(a, b, (dims, ((), ())),
                               preferred_element_type=jnp.float32)


def _mm3(a, b, dims):
    ah, al = _split(a)
    bh, bl = _split(b)
    return _mm(ah, bh, dims) + (_mm(ah, bl, dims) + _mm(al, bh, dims))


def _one_head(q, k, v, c_ref):
    qb = q.astype(jnp.bfloat16)
    kb = k.astype(jnp.bfloat16)
    run_max = jnp.full((1, L), -jnp.inf, dtype=jnp.float32)
    run_sum = jnp.zeros((1, L), dtype=jnp.float32)
    for t in range(L // KT):
        ktile = kb[t * KT:(t + 1) * KT, :]
        st = _mm(ktile, qb, ((1,), (1,)))
        cf = c_ref[t * KT:(t + 1) * KT, :].astype(jnp.float32)
        run_sum = run_sum + jnp.sum(st * cf, axis=0, keepdims=True)
        masked = jnp.where(cf > 0.0, st, -jnp.inf)
        run_max = jnp.maximum(run_max, jnp.max(masked, axis=0, keepdims=True))
    sparsity = run_max - run_sum * (1.0 / SAMPLE_K)

    iota_row = jax.lax.broadcasted_iota(jnp.int32, (1, L), 1)
    iota_col = jax.lax.broadcasted_iota(jnp.int32, (L, 1), 0)
    row40 = jax.lax.broadcasted_iota(jnp.int32, (1, N_TOP), 1)
    col40 = jax.lax.broadcasted_iota(jnp.int32, (N_TOP, 1), 0)

    def step(n, carry):
        sp, ti_row, ti_col = carry
        m = jnp.max(sp)
        idx = jnp.min(jnp.where(sp == m, iota_row, L))
        sp = jnp.where(iota_row == idx, -jnp.inf, sp)
        ti_row = jnp.where(row40 == n, idx, ti_row)
        ti_col = jnp.where(col40 == n, idx, ti_col)
        return sp, ti_row, ti_col

    _, ti_row, ti_col = jax.lax.fori_loop(
        0, N_TOP, step,
        (sparsity, jnp.zeros((1, N_TOP), jnp.int32),
         jnp.zeros((N_TOP, 1), jnp.int32)))

    p = (iota_col == ti_row).astype(jnp.float32)
    pb = p.astype(jnp.bfloat16)

    qh, ql = _split(q)
    q_top = _mm(pb, qh, ((0,), (0,))) + _mm(pb, ql, ((0,), (0,)))
    scores = _mm3(q_top, k, ((1,), (1,))) * SCALE
    key_pos = jax.lax.broadcasted_iota(jnp.int32, (N_TOP, L), 1)
    scores = jnp.where(key_pos > ti_col, -jnp.inf, scores)
    smax = jnp.max(scores, axis=1, keepdims=True)
    ex = jnp.exp(scores - smax)
    attn = ex / jnp.sum(ex, axis=1, keepdims=True)
    updates = _mm3(attn, v, ((1,), (0,)))

    ri = jax.lax.broadcasted_iota(jnp.int32, (BT, BT), 0)
    ci = jax.lax.broadcasted_iota(jnp.int32, (BT, BT), 1)
    trib = (ri >= ci).astype(jnp.bfloat16)
    vh, vl = _split(v)
    prefix = jnp.zeros((1, E), jnp.float32)
    blocks = []
    for b in range(L // BT):
        sl = slice(b * BT, (b + 1) * BT)
        cb = (_mm(trib, vh[sl], ((1,), (0,))) +
              _mm(trib, vl[sl], ((1,), (0,))) + prefix)
        blocks.append(cb)
        prefix = cb[BT - 1:BT, :]
    ctx = jnp.concatenate(blocks, axis=0)

    uh, ul = _split(updates)
    scattered = _mm(pb, uh, ((1,), (0,))) + _mm(pb, ul, ((1,), (0,)))
    is_top = jnp.sum(p, axis=1, keepdims=True) > 0.0
    return jnp.where(is_top, scattered, ctx)


def _body(q_ref, k_ref, v_ref, c_ref, o_ref):
    for i in range(2):
        sl = slice(i * E, (i + 1) * E)
        o_ref[:, sl] = _one_head(q_ref[:, sl], k_ref[:, sl], v_ref[:, sl],
                                 c_ref)


def kernel(queries, keys, values):
    B, Lq, Hn, En = queries.shape
    q2 = queries.reshape(L, H * E)
    k2 = keys.reshape(L, H * E)
    v2 = values.reshape(L, H * E)
    counts_t = jnp.asarray(_COUNTS_T)

    spec = pl.BlockSpec((L, 2 * E), lambda h: (0, h))
    spec_c = pl.BlockSpec((L, L), lambda h: (0, 0))
    out = pl.pallas_call(
        _body,
        grid=(H // 2,),
        in_specs=[spec, spec, spec, spec_c],
        out_specs=spec,
        out_shape=jax.ShapeDtypeStruct((L, H * E), jnp.float32),
        compiler_params=pltpu.CompilerParams(
            dimension_semantics=("arbitrary",)),
    )(q2, k2, v2, counts_t)
    return out.reshape(B, Lq, Hn, En)
